# Initial kernel scaffold; baseline (speedup 1.0000x reference)
import jax
import jax.numpy as jnp
from jax.experimental import pallas as pl


def kernel(logits, targets):
    raise NotImplementedError("write your pallas kernel here")


# speedup vs baseline: 1.6829x; 1.6829x over previous
import functools

import jax
import jax.numpy as jnp
import numpy as np
from jax import lax
from jax.experimental import pallas as pl
from jax.experimental.pallas import tpu as pltpu
from jax.experimental.pallas import tpu_sc as plsc

N = 16384
NS = 16
L = 16
PER = N // NS
ITERS = PER // L

CAL_W = 0.1
NBINS = 10
_BOUNDS = [np.float32(b) for b in np.linspace(0.0, 1.0, NBINS + 1)]


def _sc_body(lg_hbm, tg_hbm, out_hbm, lg_v, tg_v, accv, red_v, outv, shared):
    sid = lax.axis_index("s")
    base = sid * PER
    pltpu.sync_copy(lg_hbm.at[pl.ds(base, PER)], lg_v)
    pltpu.sync_copy(tg_hbm.at[pl.ds(base, PER)], tg_v)

    zeros = jnp.zeros((L,), jnp.float32)
    ones = jnp.ones((L,), jnp.float32)
    for k in range(3):
        accv[pl.ds(k * L, L)] = zeros

    def it(i, acc):
        logit = lg_v[pl.ds(i * L, L)]
        tgt = tg_v[pl.ds(i * L, L)]
        u = jnp.abs(logit)
        w = jnp.exp(-u)
        z = w / (w + 2.0)
        z2 = z * z
        lnm = (2.0 * z) * (1.0 + z2 * (0.33333334 + z2 * (0.2 + z2 * 0.14285715)))
        bce = jnp.maximum(logit, 0.0) - logit * tgt + lnm
        p = jnp.where(logit >= 0.0, ones, w) / (1.0 + w)
        valid = p > 0.0
        bidx = (p > _BOUNDS[1]).astype(jnp.int32)
        for b in _BOUNDS[2:NBINS]:
            bidx = bidx + (p > b).astype(jnp.int32)
        plsc.addupdate_scatter(accv, [bidx], ones, mask=valid)
        plsc.addupdate_scatter(accv, [bidx + L], tgt, mask=valid)
        plsc.addupdate_scatter(accv, [bidx + 2 * L], p, mask=valid)
        return acc + bce

    acc = lax.fori_loop(0, ITERS, it, zeros)
    accv[pl.ds(3 * L, L)] = acc

    pub_off = pl.multiple_of(sid * (4 * L), 8)
    pltpu.sync_copy(accv, shared.at[pl.ds(pub_off, 4 * L)])
    plsc.subcore_barrier()

    @pl.when(sid == 0)
    def _():
        pltpu.sync_copy(shared, red_v)

        def rit(i, carry):
            cnt, st, sp, bl = carry
            off = i * (4 * L)
            cnt = cnt + red_v[pl.ds(off, L)]
            st = st + red_v[pl.ds(off + L, L)]
            sp = sp + red_v[pl.ds(off + 2 * L, L)]
            bl = bl + red_v[pl.ds(off + 3 * L, L)]
            return cnt, st, sp, bl

        cnt, st, sp, bl = lax.fori_loop(0, NS, rit, (zeros, zeros, zeros, zeros))
        base_loss = jnp.sum(bl) * (1.0 / N)
        safe = jnp.maximum(cnt, 1.0)
        contrib = jnp.where(cnt > 0.0, jnp.abs(sp / safe - st / safe) * cnt, 0.0)
        ece = jnp.sum(contrib) * (1.0 / N)
        res = base_loss + CAL_W * ece
        outv[pl.ds(0, L)] = jnp.full((L,), res, jnp.float32)
        pltpu.sync_copy(outv, out_hbm)


@jax.jit
def _cal_loss(logits, targets):
    mesh = plsc.VectorSubcoreMesh(
        core_axis_name="c", subcore_axis_name="s", num_cores=1)
    f = pl.kernel(
        _sc_body,
        out_type=jax.ShapeDtypeStruct((L,), jnp.float32),
        mesh=mesh,
        compiler_params=pltpu.CompilerParams(needs_layout_passes=False),
        scratch_types=[
            pltpu.VMEM((PER,), jnp.float32),
            pltpu.VMEM((PER,), jnp.float32),
            pltpu.VMEM((4 * L,), jnp.float32),
            pltpu.VMEM((NS * 4 * L,), jnp.float32),
            pltpu.VMEM((L,), jnp.float32),
            pltpu.VMEM_SHARED((NS * 4 * L,), jnp.float32),
        ],
    )
    return f(logits, targets)[0]


def kernel(logits, targets):
    return _cal_loss(logits, targets)


# speedup vs baseline: 1.7278x; 1.0267x over previous
import functools

import jax
import jax.numpy as jnp
import numpy as np
from jax import lax
from jax.experimental import pallas as pl
from jax.experimental.pallas import tpu as pltpu
from jax.experimental.pallas import tpu_sc as plsc

N = 16384
NS = 16
L = 16
PER = N // NS
UNROLL = 4
ITERS = PER // (L * UNROLL)

CAL_W = 0.1
NBINS = 10
_LOG1P = [3.5110213e-06, 0.99979234, -0.49697742, 0.31458917,
          -0.18878083, 0.08172564, -0.0172078]
_BINMUL = np.float32(10.0 * (1.0 - 2.0 ** -20))


def _sc_body(lg_hbm, tg_hbm, out_hbm, lg_v, tg_v, accv, red_v, outv, shared,
             sem1, sem2):
    sid = lax.axis_index("s")
    base = sid * PER
    cp1 = pltpu.async_copy(lg_hbm.at[pl.ds(base, PER)], lg_v, sem1)
    cp2 = pltpu.async_copy(tg_hbm.at[pl.ds(base, PER)], tg_v, sem2)

    zeros = jnp.zeros((L,), jnp.float32)
    ones = jnp.ones((L,), jnp.float32)
    for k in range(3):
        accv[pl.ds(k * L, L)] = zeros
    cp1.wait()
    cp2.wait()

    def it(i, acc):
        for j in range(UNROLL):
            off = (i * UNROLL + j) * L
            logit = lg_v[pl.ds(off, L)]
            tgt = tg_v[pl.ds(off, L)]
            u = jnp.abs(logit)
            w = jnp.exp(-u)
            lnm = jnp.float32(_LOG1P[6])
            for c in _LOG1P[5::-1]:
                lnm = lnm * w + jnp.float32(c)
            acc = acc + (jnp.maximum(logit, 0.0) - logit * tgt + lnm)
            p = jnp.where(logit >= 0.0, ones, w) / (1.0 + w)
            valid = p > 0.0
            bidx = jnp.minimum((p * _BINMUL).astype(jnp.int32), NBINS - 1)
            plsc.addupdate_scatter(accv, [bidx], ones, mask=valid)
            plsc.addupdate_scatter(accv, [bidx + L], tgt, mask=valid)
            plsc.addupdate_scatter(accv, [bidx + 2 * L], p, mask=valid)
        return acc

    acc = lax.fori_loop(0, ITERS, it, zeros)
    accv[pl.ds(3 * L, L)] = acc

    pub_off = pl.multiple_of(sid * (4 * L), 8)
    pltpu.sync_copy(accv, shared.at[pl.ds(pub_off, 4 * L)])
    plsc.subcore_barrier()

    @pl.when(sid == 0)
    def _():
        pltpu.sync_copy(shared, red_v)

        def rit(i, carry):
            cnt, st, sp, bl = carry
            off = i * (4 * L)
            cnt = cnt + red_v[pl.ds(off, L)]
            st = st + red_v[pl.ds(off + L, L)]
            sp = sp + red_v[pl.ds(off + 2 * L, L)]
            bl = bl + red_v[pl.ds(off + 3 * L, L)]
            return cnt, st, sp, bl

        cnt, st, sp, bl = lax.fori_loop(0, NS, rit, (zeros, zeros, zeros, zeros))
        base_loss = jnp.sum(bl) * (1.0 / N)
        safe = jnp.maximum(cnt, 1.0)
        contrib = jnp.where(cnt > 0.0, jnp.abs(sp / safe - st / safe) * cnt, 0.0)
        ece = jnp.sum(contrib) * (1.0 / N)
        res = base_loss + CAL_W * ece
        outv[pl.ds(0, L)] = jnp.full((L,), res, jnp.float32)
        pltpu.sync_copy(outv, out_hbm)


@jax.jit
def _cal_loss(logits, targets):
    mesh = plsc.VectorSubcoreMesh(
        core_axis_name="c", subcore_axis_name="s", num_cores=1)
    f = pl.kernel(
        _sc_body,
        out_type=jax.ShapeDtypeStruct((L,), jnp.float32),
        mesh=mesh,
        compiler_params=pltpu.CompilerParams(needs_layout_passes=False),
        scratch_types=[
            pltpu.VMEM((PER,), jnp.float32),
            pltpu.VMEM((PER,), jnp.float32),
            pltpu.VMEM((4 * L,), jnp.float32),
            pltpu.VMEM((NS * 4 * L,), jnp.float32),
            pltpu.VMEM((L,), jnp.float32),
            pltpu.VMEM_SHARED((NS * 4 * L,), jnp.float32),
            pltpu.SemaphoreType.DMA,
            pltpu.SemaphoreType.DMA,
        ],
    )
    return f(logits, targets)[0]


def kernel(logits, targets):
    return _cal_loss(logits, targets)
